# Initial kernel scaffold; baseline (speedup 1.0000x reference)
import jax
import jax.numpy as jnp
from jax.experimental import pallas as pl


def kernel(x, edge_index, edge_weight, edge_attr, dim_size, W_mlp1, b_mlp1, W_mlp2, b_mlp2, W1, W2, b2, Wl, bl):
    raise NotImplementedError("write your pallas kernel here")


# speedup vs baseline: 1.3949x; 1.3949x over previous
import functools
import math

import jax
import jax.numpy as jnp
from jax import lax
from jax.experimental import pallas as pl
from jax.experimental.pallas import tpu as pltpu
from jax.experimental.pallas import tpu_sc as plsc

CUTOFF = 10.0
NC, NS, LANES = 2, 16, 16
NW = NC * NS
CH = 128


def _xh_body(x_ref, w_ref, o_ref):
    o_ref[...] = jnp.dot(x_ref[...], w_ref[...], preferred_element_type=jnp.float32)


def _filter_body(ea_ref, ew_ref, w1t_ref, b1_ref, w2t_ref, b2_ref, o_ref):
    h = jnp.dot(ea_ref[...], w1t_ref[...], preferred_element_type=jnp.float32)
    h = h + b1_ref[...]
    h = h * jax.nn.sigmoid(h)
    w = jnp.dot(h, w2t_ref[...], preferred_element_type=jnp.float32) + b2_ref[...]
    c = 0.5 * (jnp.cos(ew_ref[...] * (math.pi / CUTOFF)) + 1.0)
    o_ref[...] = w * c


def _tail_body(p_ref, w2t_ref, b2_ref, wlt_ref, bl_ref, o_ref):
    agg = p_ref[0] + p_ref[1]
    t = jnp.dot(agg, w2t_ref[...], preferred_element_type=jnp.float32) + b2_ref[...]
    t = t * jax.nn.sigmoid(t)
    o_ref[...] = jnp.dot(t, wlt_ref[...], preferred_element_type=jnp.float32) + bl_ref[...]


def _sc_body(n_pad, n_chunks, xh, wmat, src, dst, parts, sidx, didx, xrows,
             wrows, acc, sem):
    c = lax.axis_index("c")
    s = lax.axis_index("s")
    wid = s * NC + c
    rows_per_tile = n_pad // NS
    base = s * rows_per_tile

    def _zrow(i, _):
        for k in range(128 // LANES):
            xrows[i, pl.ds(k * LANES, LANES)] = jnp.zeros((LANES,), jnp.float32)
        return 0
    lax.fori_loop(0, CH, _zrow, 0)
    n_full = rows_per_tile // CH
    rem = rows_per_tile - n_full * CH
    for k in range(n_full):
        pltpu.sync_copy(xrows, acc.at[pl.ds(base + k * CH, CH)])
    if rem:
        pltpu.sync_copy(xrows.at[pl.ds(0, rem)], acc.at[pl.ds(base + n_full * CH, rem)])
    plsc.subcore_barrier()

    ebase = wid * (n_chunks * CH)

    def _chunk(j, _):
        off = ebase + j * CH
        pltpu.sync_copy(src.at[pl.ds(off, CH)], sidx)
        pltpu.sync_copy(dst.at[pl.ds(off, CH)], didx)
        pltpu.async_copy(xh.at[sidx], xrows, sem).wait()
        pltpu.sync_copy(wmat.at[pl.ds(off, CH)], wrows)

        def _mul(i, _):
            for k in range(128 // LANES):
                sl = pl.ds(k * LANES, LANES)
                xrows[i, sl] = xrows[i, sl] * wrows[i, sl]
            return 0
        lax.fori_loop(0, CH, _mul, 0)
        pltpu.sync_copy(xrows, acc.at[didx], add=True)
        return 0

    lax.fori_loop(0, n_chunks, _chunk, 0)
    plsc.subcore_barrier()

    for k in range(n_full):
        pltpu.sync_copy(acc.at[pl.ds(base + k * CH, CH)], xrows)
        pltpu.sync_copy(xrows, parts.at[c, pl.ds(base + k * CH, CH)])
    if rem:
        pltpu.sync_copy(acc.at[pl.ds(base + n_full * CH, rem)], xrows.at[pl.ds(0, rem)])
        pltpu.sync_copy(xrows.at[pl.ds(0, rem)], parts.at[c, pl.ds(base + n_full * CH, rem)])


def kernel(x, edge_index, edge_weight, edge_attr, dim_size, W_mlp1, b_mlp1,
           W_mlp2, b_mlp2, W1, W2, b2, Wl, bl):
    n, hdim = x.shape
    e = edge_index.shape[1]
    g = edge_attr.shape[1]
    f = W_mlp1.shape[0]

    ep = ((e + NW * CH - 1) // (NW * CH)) * (NW * CH)
    pad = ep - e
    src = jnp.pad(edge_index[0].astype(jnp.int32), (0, pad))
    dst = jnp.pad(edge_index[1].astype(jnp.int32), (0, pad))
    ew = jnp.pad(edge_weight, (0, pad), constant_values=CUTOFF).reshape(ep, 1)
    ea = jnp.pad(edge_attr, ((0, pad), (0, 0)))

    xh = pl.pallas_call(
        _xh_body,
        grid=(5,),
        in_specs=[
            pl.BlockSpec((n // 5, hdim), lambda i: (i, 0)),
            pl.BlockSpec((hdim, f), lambda i: (0, 0)),
        ],
        out_specs=pl.BlockSpec((n // 5, f), lambda i: (i, 0)),
        out_shape=jax.ShapeDtypeStruct((n, f), jnp.float32),
    )(x, W1.T)

    be = 1024
    wmat = pl.pallas_call(
        _filter_body,
        grid=(ep // be,),
        in_specs=[
            pl.BlockSpec((be, g), lambda i: (i, 0)),
            pl.BlockSpec((be, 1), lambda i: (i, 0)),
            pl.BlockSpec((g, f), lambda i: (0, 0)),
            pl.BlockSpec((1, f), lambda i: (0, 0)),
            pl.BlockSpec((f, f), lambda i: (0, 0)),
            pl.BlockSpec((1, f), lambda i: (0, 0)),
        ],
        out_specs=pl.BlockSpec((be, f), lambda i: (i, 0)),
        out_shape=jax.ShapeDtypeStruct((ep, f), jnp.float32),
    )(ea, ew, W_mlp1.T, b_mlp1.reshape(1, f), W_mlp2.T, b_mlp2.reshape(1, f))

    n_pad = ((n + 8 * NS - 1) // (8 * NS)) * (8 * NS)
    n_chunks = ep // (NW * CH)
    mesh = plsc.VectorSubcoreMesh(core_axis_name="c", subcore_axis_name="s")
    parts = pl.kernel(
        functools.partial(_sc_body, n_pad, n_chunks),
        out_type=jax.ShapeDtypeStruct((NC, n_pad, f), jnp.float32),
        mesh=mesh,
        scratch_types=[
            pltpu.VMEM((CH,), jnp.int32),
            pltpu.VMEM((CH,), jnp.int32),
            pltpu.VMEM((CH, f), jnp.float32),
            pltpu.VMEM((CH, f), jnp.float32),
            pltpu.VMEM_SHARED((n_pad, f), jnp.float32),
            pltpu.SemaphoreType.DMA,
        ],
    )(xh, wmat, src, dst)

    bn = n // 5
    out = pl.pallas_call(
        _tail_body,
        grid=(5,),
        in_specs=[
            pl.BlockSpec((NC, bn, f), lambda i: (0, i, 0)),
            pl.BlockSpec((f, hdim), lambda i: (0, 0)),
            pl.BlockSpec((1, hdim), lambda i: (0, 0)),
            pl.BlockSpec((hdim, hdim), lambda i: (0, 0)),
            pl.BlockSpec((1, hdim), lambda i: (0, 0)),
        ],
        out_specs=pl.BlockSpec((bn, hdim), lambda i: (i, 0)),
        out_shape=jax.ShapeDtypeStruct((n, hdim), jnp.float32),
    )(parts, W2.T, b2.reshape(1, hdim), Wl.T, bl.reshape(1, hdim))
    return out
